# baseline (speedup 1.0000x reference)
import jax
import jax.numpy as jnp
from jax.experimental import pallas as pl

N_SKILLS = 64
BATCH = 16384


def _onehot_kernel(ids_ref, out_ref):
    r = out_ref.shape[0]
    out_ref[:] = jnp.zeros((r, 1, N_SKILLS), jnp.float32)


def kernel(task_ids):
    ids2 = task_ids.reshape(128, 128).astype(jnp.int32)
    rows_per_block = 4096
    out = pl.pallas_call(
        _onehot_kernel,
        grid=(BATCH // rows_per_block,),
        in_specs=[pl.BlockSpec((32, 128), lambda i: (i, 0))],
        out_specs=pl.BlockSpec((rows_per_block, 1, N_SKILLS), lambda i: (i, 0, 0)),
        out_shape=jax.ShapeDtypeStruct((BATCH, 1, N_SKILLS), jnp.float32),
    )(ids2)
    return out


# speedup vs baseline: 4.7332x; 4.7332x over previous
import jax
import jax.numpy as jnp
from jax.experimental import pallas as pl

N_SKILLS = 64
BATCH = 16384


def _onehot_kernel(ids_ref, out_ref):
    ids = ids_ref[:]
    r = ids.shape[0]
    iota_j = jax.lax.broadcasted_iota(jnp.int32, (N_SKILLS, 128), 0)
    for k in range(r):
        row = jnp.broadcast_to(ids[k : k + 1, :], (N_SKILLS, 128))
        out_ref[:, k * 128 : (k + 1) * 128] = (row == iota_j).astype(jnp.float32)


def kernel(task_ids):
    ids2 = task_ids.reshape(128, 128).astype(jnp.int32)
    rows_per_block = 16
    out = pl.pallas_call(
        _onehot_kernel,
        grid=(128 // rows_per_block,),
        in_specs=[pl.BlockSpec((rows_per_block, 128), lambda i: (i, 0))],
        out_specs=pl.BlockSpec((N_SKILLS, rows_per_block * 128), lambda i: (0, i)),
        out_shape=jax.ShapeDtypeStruct((N_SKILLS, BATCH), jnp.float32),
    )(ids2)
    return jnp.transpose(out, (1, 0))[:, None, :]


# speedup vs baseline: 6.4768x; 1.3684x over previous
import jax
import jax.numpy as jnp
from jax.experimental import pallas as pl

N_SKILLS = 64
BATCH = 16384


def _onehot_kernel(ids_ref, out_ref):
    ids = ids_ref[:]
    r = ids.shape[0]
    iota_j = jax.lax.broadcasted_iota(jnp.int32, (N_SKILLS, 128), 0)
    for k in range(r):
        row = jnp.broadcast_to(ids[k : k + 1, :], (N_SKILLS, 128))
        out_ref[:, k * 128 : (k + 1) * 128] = (row == iota_j).astype(jnp.float32)


def kernel(task_ids):
    ids2 = task_ids.reshape(128, 128).astype(jnp.int32)
    rows_per_block = 32
    out = pl.pallas_call(
        _onehot_kernel,
        grid=(128 // rows_per_block,),
        in_specs=[pl.BlockSpec((rows_per_block, 128), lambda i: (i, 0))],
        out_specs=pl.BlockSpec((N_SKILLS, rows_per_block * 128), lambda i: (0, i)),
        out_shape=jax.ShapeDtypeStruct((N_SKILLS, BATCH), jnp.float32),
    )(ids2)
    return jnp.transpose(out, (1, 0))[:, None, :]


# speedup vs baseline: 8.1854x; 1.2638x over previous
import jax
import jax.numpy as jnp
from jax.experimental import pallas as pl

N_SKILLS = 64
BATCH = 16384


def _onehot_kernel(ids_ref, out_ref):
    ids = ids_ref[:]
    r = ids.shape[0]
    iota_j = jax.lax.broadcasted_iota(jnp.int32, (N_SKILLS, 128), 0)
    for k in range(r):
        row = jnp.broadcast_to(ids[k : k + 1, :], (N_SKILLS, 128))
        out_ref[:, k * 128 : (k + 1) * 128] = (row == iota_j).astype(jnp.float32)


def kernel(task_ids):
    ids2 = task_ids.reshape(128, 128).astype(jnp.int32)
    rows_per_block = 64
    out = pl.pallas_call(
        _onehot_kernel,
        grid=(128 // rows_per_block,),
        in_specs=[pl.BlockSpec((rows_per_block, 128), lambda i: (i, 0))],
        out_specs=pl.BlockSpec((N_SKILLS, rows_per_block * 128), lambda i: (0, i)),
        out_shape=jax.ShapeDtypeStruct((N_SKILLS, BATCH), jnp.float32),
    )(ids2)
    return jnp.transpose(out, (1, 0))[:, None, :]
